# baseline (speedup 1.0000x reference)
import functools

import jax
import jax.numpy as jnp
from jax import lax
from jax.experimental import pallas as pl
from jax.experimental.pallas import tpu as pltpu
from jax.experimental.pallas import tpu_sc as plsc

V = 4096
D = 4096
N = 16 * 2048
NC, NS, L = 2, 16, 16
NW = NC * NS
TPW = N // NW
C = 4
NBUF = 6
AHEAD = NBUF - 2
NCH = TPW // C
RB = 256
IDX_CHUNK = 128


def _lse_block(x_ref, o_ref):
    i = pl.program_id(0)
    x = x_ref[...]
    m = jnp.max(x, axis=1, keepdims=True)
    s = jnp.sum(jnp.exp(x - m), axis=1)
    o_ref[0, pl.ds(i * RB, RB)] = m[:, 0] + jnp.log(s)


def _finalize_block(p_ref, o_ref):
    o_ref[...] = jnp.sum(p_ref[...]).reshape(1, 1) * (1.0 / N)


_MESH = plsc.VectorSubcoreMesh(core_axis_name="c", subcore_axis_name="s")


@functools.partial(
    pl.kernel,
    out_type=jax.ShapeDtypeStruct((N, 32, 128), jnp.float32),
    mesh=_MESH,
    scratch_types=[
        pltpu.VMEM((2 * TPW,), jnp.int32),
        pltpu.VMEM((NBUF, C, 32, 128), jnp.float32),
        pltpu.SemaphoreType.DMA,
        pltpu.SemaphoreType.DMA((NBUF,)),
        pltpu.SemaphoreType.DMA((NBUF,)),
    ],
)
def _sc_rows(tlin, idxp, out, idxp_v, bufs, sem_t, gsems, psems):
    wid = lax.axis_index("s") * NC + lax.axis_index("c")
    base = wid * TPW
    pltpu.async_copy(idxp.at[pl.ds(2 * base, 2 * TPW)], idxp_v, sem_t).wait()

    def g_start(ch, b):
        pltpu.async_copy(tlin.at[idxp_v.at[pl.ds(ch * 8, C)]],
                         bufs.at[b], gsems.at[b])

    def g_wait(b):
        pltpu.make_async_copy(tlin.at[pl.ds(0, C)], bufs.at[b],
                              gsems.at[b]).wait()

    def p_start(ch, b):
        pltpu.async_copy(bufs.at[b], out.at[pl.ds(base + ch * C, C)],
                         psems.at[b])

    def p_wait(b):
        pltpu.make_async_copy(bufs.at[b], out.at[pl.ds(base, C)],
                              psems.at[b]).wait()

    for ch in range(AHEAD):
        g_start(ch, ch)

    for b in range(NBUF):
        g_wait(b)
        p_start(b, b)
        nb = (b + AHEAD) % NBUF
        if b + AHEAD >= NBUF:
            p_wait(nb)
        g_start(b + AHEAD, nb)

    def ring(g0, carry):
        for b in range(NBUF):
            ch = g0 * NBUF + b
            g_wait(b)
            p_start(ch, b)

            @pl.when(ch + AHEAD < NCH)
            def _():
                nb = (b + AHEAD) % NBUF
                p_wait(nb)
                g_start(ch + AHEAD, nb)

        return carry

    lax.fori_loop(1, NCH // NBUF, ring, 0)
    for ch in range(NCH - NCH % NBUF, NCH):
        g_wait(ch % NBUF)
        p_start(ch, ch % NBUF)
    for b in range(NBUF):
        p_wait(b)


@functools.partial(
    pl.kernel,
    out_type=jax.ShapeDtypeStruct((NW, L), jnp.float32),
    mesh=_MESH,
    scratch_types=[
        pltpu.VMEM((TPW,), jnp.int32),
        pltpu.VMEM((TPW,), jnp.int32),
        pltpu.VMEM((TPW,), jnp.int32),
        pltpu.VMEM((TPW,), jnp.float32),
        pltpu.VMEM((TPW,), jnp.float32),
        pltpu.VMEM((L,), jnp.float32),
        pltpu.SemaphoreType.DMA,
    ],
)
def _sc_loss(tphys, idxf, tgtf, lse, parts,
             idx_v, tgt_v, flat_v, tval_v, lse_tok_v, pstage, sem_t):
    wid = lax.axis_index("s") * NC + lax.axis_index("c")
    base = wid * TPW
    cp_i = pltpu.async_copy(idxf.at[pl.ds(base, TPW)], idx_v, sem_t)
    cp_t = pltpu.async_copy(tgtf.at[pl.ds(base, TPW)], tgt_v, sem_t)
    cp_i.wait()
    cp_t.wait()

    def fbody(g, carry):
        i16 = idx_v[pl.ds(g * L, L)]
        t16 = tgt_v[pl.ds(g * L, L)]
        flat_v[pl.ds(g * L, L)] = i16 * D + t16
        return carry

    lax.fori_loop(0, TPW // L, fbody, 0)

    tcopies = [
        pltpu.async_copy(
            tphys.at[flat_v.at[pl.ds(j * IDX_CHUNK, IDX_CHUNK)]],
            tval_v.at[pl.ds(j * IDX_CHUNK, IDX_CHUNK)],
            sem_t,
        )
        for j in range(TPW // IDX_CHUNK)
    ] + [
        pltpu.async_copy(
            lse.at[idx_v.at[pl.ds(j * IDX_CHUNK, IDX_CHUNK)]],
            lse_tok_v.at[pl.ds(j * IDX_CHUNK, IDX_CHUNK)],
            sem_t,
        )
        for j in range(TPW // IDX_CHUNK)
    ]
    for cp in tcopies:
        cp.wait()

    def lbody(g, acc):
        l16 = lse_tok_v[pl.ds(g * L, L)]
        return acc + (l16 - tval_v[pl.ds(g * L, L)])

    acc = lax.fori_loop(0, TPW // L, lbody, jnp.zeros((L,), jnp.float32))
    pstage[...] = acc
    pltpu.sync_copy(pstage, parts.at[wid])


def kernel(idx, targets, token_embedding_table):
    table = token_embedding_table
    idxf = idx.reshape(N).astype(jnp.int32)
    tgtf = targets.reshape(N).astype(jnp.int32)

    lse2 = pl.pallas_call(
        _lse_block,
        grid=(V // RB,),
        in_specs=[pl.BlockSpec((RB, D), lambda i: (i, 0))],
        out_specs=pl.BlockSpec((1, V), lambda i: (0, 0)),
        out_shape=jax.ShapeDtypeStruct((1, V), jnp.float32),
    )(table)
    lse = lse2.reshape(V)

    idxp = jnp.pad(idxf.reshape(N // C, C), ((0, 0), (0, 8 - C))).reshape(-1)

    out4 = _sc_rows(table.reshape(V, 32, 128), idxp)
    logits = out4.reshape(N, D)
    parts = _sc_loss(table.reshape(V * D), idxf, tgtf, lse)

    loss2 = pl.pallas_call(
        _finalize_block,
        out_shape=jax.ShapeDtypeStruct((1, 1), jnp.float32),
    )(parts)
    return logits, loss2.reshape(())


# speedup vs baseline: 1.9841x; 1.9841x over previous
import functools

import jax
import jax.numpy as jnp
from jax import lax
from jax.experimental import pallas as pl
from jax.experimental.pallas import tpu as pltpu
from jax.experimental.pallas import tpu_sc as plsc

V = 4096
D = 4096
N = 16 * 2048
NC, NS, L = 2, 16, 16
NW = NC * NS
TPW = N // NW
C = 8
NBUF = 3
NCH = TPW // C
RB = 256
IDX_CHUNK = 128


def _lse_block(x_ref, o_ref):
    i = pl.program_id(0)
    x = x_ref[...]
    m = jnp.max(x, axis=1, keepdims=True)
    s = jnp.sum(jnp.exp(x - m), axis=1)
    o_ref[0, pl.ds(i * RB, RB)] = m[:, 0] + jnp.log(s)


def _finalize_block(p_ref, o_ref):
    o_ref[...] = jnp.sum(p_ref[...]).reshape(1, 1) * (1.0 / N)


_MESH = plsc.VectorSubcoreMesh(core_axis_name="c", subcore_axis_name="s")


@functools.partial(
    pl.kernel,
    out_type=[
        jax.ShapeDtypeStruct((N, D), jnp.float32),
        jax.ShapeDtypeStruct((NW, L), jnp.float32),
    ],
    mesh=_MESH,
    scratch_types=[
        pltpu.VMEM((TPW,), jnp.int32),
        pltpu.VMEM((TPW,), jnp.int32),
        pltpu.VMEM((TPW,), jnp.int32),
        pltpu.VMEM((TPW,), jnp.float32),
        pltpu.VMEM((TPW,), jnp.float32),
        pltpu.VMEM((NBUF, C, D), jnp.float32),
        pltpu.VMEM((L,), jnp.float32),
        pltpu.SemaphoreType.DMA,
        pltpu.SemaphoreType.DMA((NBUF,)),
        pltpu.SemaphoreType.DMA((NBUF,)),
    ],
)
def _sc_gather_loss(table, tflat, idxf, tgtf, lse, out, parts,
                    idx_v, tgt_v, flat_v, tval_v, lse_tok_v, bufs, pstage,
                    sem_t, gsems, psems):
    wid = lax.axis_index("s") * NC + lax.axis_index("c")
    base = wid * TPW
    cp_i = pltpu.async_copy(idxf.at[pl.ds(base, TPW)], idx_v, sem_t)
    cp_t = pltpu.async_copy(tgtf.at[pl.ds(base, TPW)], tgt_v, sem_t)
    cp_i.wait()
    cp_t.wait()

    def g_start(ch, b):
        pltpu.async_copy(table.at[idx_v.at[pl.ds(ch * C, C)]],
                         bufs.at[b], gsems.at[b])

    def g_wait(b):
        pltpu.make_async_copy(table.at[pl.ds(0, C)], bufs.at[b],
                              gsems.at[b]).wait()

    def p_start(ch, b):
        pltpu.async_copy(bufs.at[b], out.at[pl.ds(base + ch * C, C)],
                         psems.at[b])

    def p_wait(b):
        pltpu.make_async_copy(bufs.at[b], out.at[pl.ds(base, C)],
                              psems.at[b]).wait()

    for b in range(NBUF):
        g_start(b, b)

    def fbody(g, carry):
        i16 = idx_v[pl.ds(g * L, L)]
        t16 = tgt_v[pl.ds(g * L, L)]
        flat_v[pl.ds(g * L, L)] = i16 * D + t16
        return carry

    lax.fori_loop(0, TPW // L, fbody, 0)

    tcopies = [
        pltpu.async_copy(
            tflat.at[flat_v.at[pl.ds(j * IDX_CHUNK, IDX_CHUNK)]],
            tval_v.at[pl.ds(j * IDX_CHUNK, IDX_CHUNK)],
            sem_t,
        )
        for j in range(TPW // IDX_CHUNK)
    ] + [
        pltpu.async_copy(
            lse.at[idx_v.at[pl.ds(j * IDX_CHUNK, IDX_CHUNK)]],
            lse_tok_v.at[pl.ds(j * IDX_CHUNK, IDX_CHUNK)],
            sem_t,
        )
        for j in range(TPW // IDX_CHUNK)
    ]
    for cp in tcopies:
        cp.wait()

    def lbody(g, acc):
        l16 = lse_tok_v[pl.ds(g * L, L)]
        return acc + (l16 - tval_v[pl.ds(g * L, L)])

    acc = lax.fori_loop(0, TPW // L, lbody, jnp.zeros((L,), jnp.float32))
    pstage[...] = acc
    pltpu.sync_copy(pstage, parts.at[wid])

    def ring(g0, carry):
        for b in range(NBUF):
            ch = g0 * NBUF + b
            g_wait(b)
            p_start(ch, b)

            @pl.when(ch + NBUF < NCH)
            def _():
                p_wait(b)
                g_start(ch + NBUF, b)

        return carry

    lax.fori_loop(0, NCH // NBUF, ring, 0)
    for ch in range(NCH - NCH % NBUF, NCH):
        g_wait(ch % NBUF)
        p_start(ch, ch % NBUF)
    for b in range(NBUF):
        p_wait(b)


def kernel(idx, targets, token_embedding_table):
    table = token_embedding_table
    idxf = idx.reshape(N).astype(jnp.int32)
    tgtf = targets.reshape(N).astype(jnp.int32)

    lse2 = pl.pallas_call(
        _lse_block,
        grid=(V // RB,),
        in_specs=[pl.BlockSpec((RB, D), lambda i: (i, 0))],
        out_specs=pl.BlockSpec((1, V), lambda i: (0, 0)),
        out_shape=jax.ShapeDtypeStruct((1, V), jnp.float32),
    )(table)
    lse = lse2.reshape(V)

    logits, parts = _sc_gather_loss(table, table.reshape(V * D), idxf, tgtf,
                                    lse)

    loss2 = pl.pallas_call(
        _finalize_block,
        out_shape=jax.ShapeDtypeStruct((1, 1), jnp.float32),
    )(parts)
    return logits, loss2.reshape(())


# speedup vs baseline: 1.9842x; 1.0001x over previous
import functools

import jax
import jax.numpy as jnp
from jax import lax
from jax.experimental import pallas as pl
from jax.experimental.pallas import tpu as pltpu
from jax.experimental.pallas import tpu_sc as plsc

V = 4096
D = 4096
N = 16 * 2048
NC, NS, L = 2, 16, 16
NW = NC * NS
TPW = N // NW
C = 8
NBUF = 3
NCH = TPW // C
RB = 256
IDX_CHUNK = 128


def _lse_block(x_ref, o_ref):
    i = pl.program_id(0)
    x = x_ref[...]
    m = jnp.max(x, axis=1, keepdims=True)
    s = jnp.sum(jnp.exp(x - m), axis=1)
    o_ref[0, pl.ds(i * RB, RB)] = m[:, 0] + jnp.log(s)


def _finalize_block(p_ref, o_ref):
    o_ref[...] = jnp.sum(p_ref[...]).reshape(1, 1) * (1.0 / N)


_MESH = plsc.VectorSubcoreMesh(core_axis_name="c", subcore_axis_name="s")


@functools.partial(
    pl.kernel,
    out_type=[
        jax.ShapeDtypeStruct((N, D), jnp.float32),
        jax.ShapeDtypeStruct((NW, L), jnp.float32),
    ],
    mesh=_MESH,
    scratch_types=[
        pltpu.VMEM((TPW,), jnp.int32),
        pltpu.VMEM((TPW,), jnp.int32),
        pltpu.VMEM((TPW,), jnp.int32),
        pltpu.VMEM((TPW,), jnp.float32),
        pltpu.VMEM((TPW,), jnp.float32),
        pltpu.VMEM((NBUF, C, D), jnp.float32),
        pltpu.VMEM((L,), jnp.float32),
        pltpu.SemaphoreType.DMA,
        pltpu.SemaphoreType.DMA,
        pltpu.SemaphoreType.DMA,
        pltpu.SemaphoreType.DMA,
        pltpu.SemaphoreType.DMA,
        pltpu.SemaphoreType.DMA,
        pltpu.SemaphoreType.DMA,
    ],
)
def _sc_gather_loss(table, tflat, idxf, tgtf, lse, out, parts,
                    idx_v, tgt_v, flat_v, tval_v, lse_tok_v, bufs, pstage,
                    sem_t, gsem0, gsem1, gsem2, psem0, psem1, psem2):
    gsems = (gsem0, gsem1, gsem2)
    psems = (psem0, psem1, psem2)
    wid = lax.axis_index("s") * NC + lax.axis_index("c")
    base = wid * TPW
    cp_i = pltpu.async_copy(idxf.at[pl.ds(base, TPW)], idx_v, sem_t)
    cp_t = pltpu.async_copy(tgtf.at[pl.ds(base, TPW)], tgt_v, sem_t)
    cp_i.wait()
    cp_t.wait()

    def g_start(ch, b):
        pltpu.async_copy(table.at[idx_v.at[pl.ds(ch * C, C)]],
                         bufs.at[b], gsems[b])

    def g_wait(b):
        pltpu.make_async_copy(table.at[pl.ds(0, C)], bufs.at[b],
                              gsems[b]).wait()

    def p_start(ch, b):
        pltpu.async_copy(bufs.at[b], out.at[pl.ds(base + ch * C, C)],
                         psems[b])

    def p_wait(b):
        pltpu.make_async_copy(bufs.at[b], out.at[pl.ds(base, C)],
                              psems[b]).wait()

    for b in range(NBUF):
        g_start(b, b)

    def fbody(g, carry):
        i16 = idx_v[pl.ds(g * L, L)]
        t16 = tgt_v[pl.ds(g * L, L)]
        flat_v[pl.ds(g * L, L)] = i16 * D + t16
        return carry

    lax.fori_loop(0, TPW // L, fbody, 0)

    tcopies = [
        pltpu.async_copy(
            tflat.at[flat_v.at[pl.ds(j * IDX_CHUNK, IDX_CHUNK)]],
            tval_v.at[pl.ds(j * IDX_CHUNK, IDX_CHUNK)],
            sem_t,
        )
        for j in range(TPW // IDX_CHUNK)
    ] + [
        pltpu.async_copy(
            lse.at[idx_v.at[pl.ds(j * IDX_CHUNK, IDX_CHUNK)]],
            lse_tok_v.at[pl.ds(j * IDX_CHUNK, IDX_CHUNK)],
            sem_t,
        )
        for j in range(TPW // IDX_CHUNK)
    ]
    for cp in tcopies:
        cp.wait()

    def lbody(g, acc):
        l16 = lse_tok_v[pl.ds(g * L, L)]
        return acc + (l16 - tval_v[pl.ds(g * L, L)])

    acc = lax.fori_loop(0, TPW // L, lbody, jnp.zeros((L,), jnp.float32))
    pstage[...] = acc
    pltpu.sync_copy(pstage, parts.at[wid])

    def ring(g0, carry):
        for b in range(NBUF):
            ch = g0 * NBUF + b
            g_wait(b)
            p_start(ch, b)

            @pl.when(ch + NBUF < NCH)
            def _():
                p_wait(b)
                g_start(ch + NBUF, b)

        return carry

    lax.fori_loop(0, NCH // NBUF, ring, 0)
    for ch in range(NCH - NCH % NBUF, NCH):
        g_wait(ch % NBUF)
        p_start(ch, ch % NBUF)
    for b in range(NBUF):
        p_wait(b)


def kernel(idx, targets, token_embedding_table):
    table = token_embedding_table
    idxf = idx.reshape(N).astype(jnp.int32)
    tgtf = targets.reshape(N).astype(jnp.int32)

    lse2 = pl.pallas_call(
        _lse_block,
        grid=(V // RB,),
        in_specs=[pl.BlockSpec((RB, D), lambda i: (i, 0))],
        out_specs=pl.BlockSpec((1, V), lambda i: (0, 0)),
        out_shape=jax.ShapeDtypeStruct((1, V), jnp.float32),
    )(table)
    lse = lse2.reshape(V)

    logits, parts = _sc_gather_loss(table, table.reshape(V * D), idxf, tgtf,
                                    lse)

    loss2 = pl.pallas_call(
        _finalize_block,
        out_shape=jax.ShapeDtypeStruct((1, 1), jnp.float32),
    )(parts)
    return logits, loss2.reshape(())
